# Initial kernel scaffold; baseline (speedup 1.0000x reference)
import jax
import jax.numpy as jnp
from jax.experimental import pallas as pl


def kernel(y, tables, W, b):
    raise NotImplementedError("write your pallas kernel here")


# speedup vs baseline: 15.8749x; 15.8749x over previous
import functools

import jax
import jax.numpy as jnp
from jax import lax
from jax.experimental import pallas as pl
from jax.experimental.pallas import tpu as pltpu
from jax.experimental.pallas import tpu_sc as plsc

_VOCAB = 100000
_F = 8
_D = 16
_DM = 128
_NC, _NS, _LANES = 2, 16, 16
_NW = _NC * _NS

_CHUNK = 2048


def _sc_gather(y_flat, tables_flat):
    total = y_flat.shape[0]
    per_w = total // _NW
    nch = per_w // _CHUNK
    assert per_w % _CHUNK == 0

    mesh = plsc.VectorSubcoreMesh(
        core_axis_name="c", subcore_axis_name="s",
        num_cores=_NC, num_subcores=_NS)

    @functools.partial(
        pl.kernel,
        out_type=jax.ShapeDtypeStruct((total, _D), jnp.float32),
        mesh=mesh,
        scratch_types=[
            pltpu.VMEM((_CHUNK,), jnp.int32),
            pltpu.VMEM((_CHUNK,), jnp.int32),
            pltpu.VMEM((_CHUNK, _D), jnp.float32),
            pltpu.SemaphoreType.DMA,
        ],
        compiler_params=pltpu.CompilerParams(use_tc_tiling_on_sc=False),
    )
    def k(y_hbm, tab_hbm, out_hbm, y_v, idx_v, rows_v, sem):
        wid = lax.axis_index("s") * _NC + lax.axis_index("c")
        base = wid * per_w
        offvec = (lax.iota(jnp.int32, _LANES) % _F) * _VOCAB

        def chunk_body(ci, carry):
            off = base + ci * _CHUNK
            pltpu.sync_copy(y_hbm.at[pl.ds(off, _CHUNK)], y_v)

            def vec_body(j, c2):
                s = pl.multiple_of(j * _LANES, _LANES)
                idx_v[pl.ds(s, _LANES)] = y_v[pl.ds(s, _LANES)] + offvec
                return c2

            lax.fori_loop(0, _CHUNK // _LANES, vec_body, 0)
            pltpu.async_copy(tab_hbm.at[idx_v], rows_v, sem).wait()
            pltpu.sync_copy(rows_v, out_hbm.at[pl.ds(off, _CHUNK)])
            return carry

        lax.fori_loop(0, nch, chunk_body, 0)

    return k(y_flat, tables_flat)


def _tc_project(x, w, b2):
    m = x.shape[0]
    bm = 2048
    assert m % bm == 0

    def body(x_ref, w_ref, b_ref, o_ref):
        o_ref[...] = lax.dot_general(
            x_ref[...], w_ref[...], (((1,), (1,)), ((), ())),
            preferred_element_type=jnp.float32) + b_ref[...]

    return pl.pallas_call(
        body,
        grid=(m // bm,),
        in_specs=[
            pl.BlockSpec((bm, _DM), lambda i: (i, 0)),
            pl.BlockSpec((_DM, _DM), lambda i: (0, 0)),
            pl.BlockSpec((1, _DM), lambda i: (0, 0)),
        ],
        out_specs=pl.BlockSpec((bm, _DM), lambda i: (i, 0)),
        out_shape=jax.ShapeDtypeStruct((m, _DM), jnp.float32),
    )(x, w, b2)


def kernel(y, tables, W, b):
    bb, ll, ff = y.shape
    y_flat = y.reshape(-1)
    tab_flat = tables.reshape(_F * _VOCAB, _D)
    gathered = _sc_gather(y_flat, tab_flat)
    x = gathered.reshape(bb * ll, _F * _D)
    out = _tc_project(x, W, b.reshape(1, _DM))
    return out.reshape(bb, ll, _DM)
